# baseline (speedup 1.0000x reference)
import functools

import jax
import jax.numpy as jnp
from jax import lax
from jax.experimental import pallas as pl
from jax.experimental.pallas import tpu as pltpu
from jax.experimental.pallas import tpu_sc as plsc

VOCAB = 1000000
MAX_LEN = 200
D = 64
EPS = 1e-5

NC = 2
NS = 16
NW = NC * NS
BATCH = 4096
NTOK = BATCH * MAX_LEN
TOK_PER_W = NTOK // NW
CH = 128
STEPS = TOK_PER_W // CH
NBUF = 4


def _shuffle(s, perm):
    return lax.gather(
        s,
        perm,
        dimension_numbers=lax.GatherDimensionNumbers(
            offset_dims=(), collapsed_slice_dims=(0,), start_index_map=(0,)
        ),
        slice_sizes=(1,),
        mode=lax.GatherScatterMode.PROMISE_IN_BOUNDS,
    )


def _xsum(s):
    lanes = lax.iota(jnp.int32, 16)
    for k in (8, 4, 2, 1):
        perm = jnp.reshape(lanes ^ k, (16, 1))
        s = s + _shuffle(s, perm)
    return s


def _rsqrt_vec(x):
    i = lax.bitcast_convert_type(x, jnp.int32)
    i = jnp.int32(0x5F3759DF) - lax.shift_right_logical(i, 1)
    y = lax.bitcast_convert_type(i, jnp.float32)
    for _ in range(3):
        y = y * (jnp.float32(1.5) - jnp.float32(0.5) * x * y * y)
    return y


def _ln_body(v):
    sq = [vk * vk for vk in v]
    mean = _xsum((v[0] + v[1]) + (v[2] + v[3])) * jnp.float32(1.0 / 64.0)
    ex2 = _xsum((sq[0] + sq[1]) + (sq[2] + sq[3])) * jnp.float32(1.0 / 64.0)
    var = ex2 - mean * mean
    rs = _rsqrt_vec(var + jnp.float32(EPS))
    return [(vk - mean) * rs for vk in v]


@functools.partial(
    pl.kernel,
    mesh=plsc.VectorSubcoreMesh(core_axis_name="c", subcore_axis_name="s"),
    compiler_params=pltpu.CompilerParams(use_tc_tiling_on_sc=False),
    out_type=jax.ShapeDtypeStruct((NTOK, D), jnp.float32),
    scratch_types=(
        [pltpu.VMEM((TOK_PER_W,), jnp.int32)]
        + [pltpu.VMEM((CH, D), jnp.float32)] * NBUF
        + [pltpu.VMEM((CH, D), jnp.float32)] * NBUF
        + [pltpu.VMEM((MAX_LEN, D), jnp.float32)]
        + [pltpu.SemaphoreType.DMA] * (2 * NBUF)
    ),
)
def _emb_body(x_hbm, tok_hbm, pos_hbm, out_hbm, idx_v, *scratch):
    in_b = scratch[0:NBUF]
    out_b = scratch[NBUF:2 * NBUF]
    pos_v = scratch[2 * NBUF]
    sem_g = scratch[2 * NBUF + 1:2 * NBUF + 1 + NBUF]
    sem_w = scratch[2 * NBUF + 1 + NBUF:2 * NBUF + 1 + 2 * NBUF]

    wid = lax.axis_index("s") * NC + lax.axis_index("c")
    base = pl.multiple_of(wid * TOK_PER_W, TOK_PER_W)
    pltpu.sync_copy(pos_hbm, pos_v)
    pltpu.sync_copy(x_hbm.at[pl.ds(base, TOK_PER_W)], idx_v)

    def gather(g, b):
        pltpu.async_copy(
            tok_hbm.at[idx_v.at[pl.ds(g * CH, CH)]], in_b[b], sem_g[b]
        )

    for b in range(NBUF):
        gather(b, b)

    @pl.loop(0, STEPS, step=NBUF)
    def outer(g0):
        for b in range(NBUF):
            g = g0 + b
            off = g * CH
            pltpu.make_async_copy(
                tok_hbm.at[idx_v.at[pl.ds(off, CH)]], in_b[b], sem_g[b]
            ).wait()
            @pl.when(g0 > 0)
            def _():
                pltpu.make_async_copy(
                    out_b[b], out_hbm.at[pl.ds(base + off - NBUF * CH, CH)],
                    sem_w[b],
                ).wait()

            s_pos = (base + off) // BATCH
            pr = [pos_v[s_pos, pl.ds(16 * k, 16)] for k in range(4)]

            @plsc.parallel_loop(0, CH, unroll=8)
            def token(r):
                v = [
                    in_b[b][r, pl.ds(16 * k, 16)] + pr[k] for k in range(4)
                ]
                o = _ln_body(v)
                for k in range(4):
                    out_b[b][r, pl.ds(16 * k, 16)] = o[k]

            @pl.when(g + NBUF < STEPS)
            def _():
                gather(g + NBUF, b)

            pltpu.async_copy(
                out_b[b], out_hbm.at[pl.ds(base + off, CH)], sem_w[b]
            )

    for b in range(NBUF):
        off = (STEPS - NBUF + b) * CH
        pltpu.make_async_copy(
            out_b[b], out_hbm.at[pl.ds(base + off, CH)], sem_w[b]
        ).wait()


def kernel(x, tok_table, pos_table):
    b, s = x.shape
    xs = x.T.reshape(-1).astype(jnp.int32)
    out = _emb_body(xs, tok_table, pos_table)
    return out.reshape(s, b, D).transpose(1, 0, 2)


# speedup vs baseline: 1.0297x; 1.0297x over previous
import functools

import jax
import jax.numpy as jnp
from jax import lax
from jax.experimental import pallas as pl
from jax.experimental.pallas import tpu as pltpu
from jax.experimental.pallas import tpu_sc as plsc

VOCAB = 1000000
MAX_LEN = 200
D = 64
EPS = 1e-5

NC = 2
NS = 16
NW = NC * NS
BATCH = 4096
NTOK = BATCH * MAX_LEN
TOK_PER_W = NTOK // NW
CH = 128
STEPS = TOK_PER_W // CH
NBUF = 4


def _shuffle(s, perm):
    return lax.gather(
        s,
        perm,
        dimension_numbers=lax.GatherDimensionNumbers(
            offset_dims=(), collapsed_slice_dims=(0,), start_index_map=(0,)
        ),
        slice_sizes=(1,),
        mode=lax.GatherScatterMode.PROMISE_IN_BOUNDS,
    )


def _xsum(s):
    lanes = lax.iota(jnp.int32, 16)
    for k in (8, 4, 2, 1):
        perm = jnp.reshape(lanes ^ k, (16, 1))
        s = s + _shuffle(s, perm)
    return s


def _rsqrt_vec(x):
    i = lax.bitcast_convert_type(x, jnp.int32)
    i = jnp.int32(0x5F3759DF) - lax.shift_right_logical(i, 1)
    y = lax.bitcast_convert_type(i, jnp.float32)
    for _ in range(3):
        y = y * (jnp.float32(1.5) - jnp.float32(0.5) * x * y * y)
    return y


def _ln_body(v):
    sq = [vk * vk for vk in v]
    mean = _xsum((v[0] + v[1]) + (v[2] + v[3])) * jnp.float32(1.0 / 64.0)
    ex2 = _xsum((sq[0] + sq[1]) + (sq[2] + sq[3])) * jnp.float32(1.0 / 64.0)
    var = ex2 - mean * mean
    rs = _rsqrt_vec(var + jnp.float32(EPS))
    return [(vk - mean) * rs for vk in v]


@functools.partial(
    pl.kernel,
    mesh=plsc.VectorSubcoreMesh(core_axis_name="c", subcore_axis_name="s"),
    compiler_params=pltpu.CompilerParams(use_tc_tiling_on_sc=False),
    out_type=jax.ShapeDtypeStruct((NTOK, D), jnp.float32),
    scratch_types=(
        [pltpu.VMEM((TOK_PER_W,), jnp.int32)]
        + [pltpu.VMEM((CH, D), jnp.float32)] * NBUF
        + [pltpu.VMEM((CH, D), jnp.float32)] * NBUF
        + [pltpu.VMEM((MAX_LEN, D), jnp.float32)]
        + [pltpu.SemaphoreType.DMA] * (2 * NBUF)
    ),
)
def _emb_body(x_hbm, tok_hbm, pos_hbm, out_hbm, idx_v, *scratch):
    in_b = scratch[0:NBUF]
    out_b = scratch[NBUF:2 * NBUF]
    pos_v = scratch[2 * NBUF]
    sem_g = scratch[2 * NBUF + 1:2 * NBUF + 1 + NBUF]
    sem_w = scratch[2 * NBUF + 1 + NBUF:2 * NBUF + 1 + 2 * NBUF]

    wid = lax.axis_index("s") * NC + lax.axis_index("c")
    base = pl.multiple_of(wid * TOK_PER_W, TOK_PER_W)
    pltpu.sync_copy(pos_hbm, pos_v)
    pltpu.sync_copy(x_hbm.at[pl.ds(base, TOK_PER_W)], idx_v)

    def gather(g, b):
        pltpu.async_copy(
            tok_hbm.at[idx_v.at[pl.ds(g * CH, CH)]], in_b[b], sem_g[b]
        )

    for b in range(NBUF):
        gather(b, b)

    @pl.loop(0, STEPS, step=NBUF)
    def outer(g0):
        for b in range(NBUF):
            g = g0 + b
            off = g * CH
            pltpu.make_async_copy(
                tok_hbm.at[idx_v.at[pl.ds(off, CH)]], in_b[b], sem_g[b]
            ).wait()
            @pl.when(g0 > 0)
            def _():
                pltpu.make_async_copy(
                    out_b[b], out_hbm.at[pl.ds(base + off - NBUF * CH, CH)],
                    sem_w[b],
                ).wait()

            s_pos = (base + off) // BATCH
            pr = [pos_v[s_pos, pl.ds(16 * k, 16)] for k in range(4)]

            @plsc.parallel_loop(0, CH, unroll=4)
            def token(r):
                v = [
                    in_b[b][r, pl.ds(16 * k, 16)] + pr[k] for k in range(4)
                ]
                o = _ln_body(v)
                for k in range(4):
                    out_b[b][r, pl.ds(16 * k, 16)] = o[k]

            @pl.when(g + NBUF < STEPS)
            def _():
                gather(g + NBUF, b)

            pltpu.async_copy(
                out_b[b], out_hbm.at[pl.ds(base + off, CH)], sem_w[b]
            )

    for b in range(NBUF):
        off = (STEPS - NBUF + b) * CH
        pltpu.make_async_copy(
            out_b[b], out_hbm.at[pl.ds(base + off, CH)], sem_w[b]
        ).wait()


def kernel(x, tok_table, pos_table):
    b, s = x.shape
    xs = x.T.reshape(-1).astype(jnp.int32)
    out = _emb_body(xs, tok_table, pos_table)
    return out.reshape(s, b, D).transpose(1, 0, 2)
